# baseline (speedup 1.0000x reference)
import jax
import jax.numpy as jnp
from jax import lax
from jax.experimental import pallas as pl
from jax.experimental.pallas import tpu as pltpu
from jax.experimental.pallas import tpu_sc as plsc

N_NODES = 10000
N_EDGES = 320000
D_FEAT = 128
N_TARGETS = 1024

NC = 2
NS = 16
CHUNK = 128
E_TILE = N_EDGES // (NC * NS)
DUMP_SLOT = N_TARGETS
AGG_ROWS = 1280
SL_ROWS = 80
INV_ROWS = N_NODES + 16


def _sc_body(nf_hbm, src_hbm, dst_hbm, tgt_hbm, o_hbm,
             inv_v, src_v, dst_v, slots2, srcs2, rows_v0, rows_v1,
             zb_v, tgt_v, ob_v, agg_sh, inv_sh, gsem0, gsem1):
    c = lax.axis_index("c")
    s = lax.axis_index("s")

    base = (c * NS + s) * E_TILE
    with jax.named_scope("stage_idx"):
        pltpu.sync_copy(src_hbm.at[pl.ds(base, E_TILE)], src_v)
        pltpu.sync_copy(dst_hbm.at[pl.ds(base, E_TILE)], dst_v)

    for i in range(16):
        for k in range(D_FEAT // 16):
            zb_v[i, pl.ds(k * 16, 16)] = jnp.zeros((16,), jnp.float32)
    for k in range(5):
        pltpu.sync_copy(zb_v, agg_sh.at[pl.ds(s * 80 + k * 16, 16)])

    @pl.when(s == 0)
    def _():
        def clr(i, carry):
            inv_v[pl.ds(i * 16, 16)] = jnp.full((16,), -1, jnp.int32)
            return carry
        lax.fori_loop(0, INV_ROWS // 16, clr, 0)
        pltpu.sync_copy(tgt_hbm, tgt_v)
        def bld(j, carry):
            tv = tgt_v[pl.ds(j * 16, 16)]
            vals = lax.iota(jnp.int32, 16) + j * 16
            plsc.store_scatter(inv_v, [tv], vals)
            return carry
        lax.fori_loop(0, N_TARGETS // 16, bld, 0)
        pltpu.sync_copy(inv_v, inv_sh)

    plsc.subcore_barrier()
    with jax.named_scope("inv_fetch"):
        pltpu.sync_copy(inv_sh, inv_v)

    lane15 = jnp.full((16,), 15, jnp.int32)

    def filt(i, ptr):
        dvec = dst_v[pl.ds(i * 16, 16)]
        svec = src_v[pl.ds(i * 16, 16)]
        slot = plsc.load_gather(inv_v, [dvec])
        mask = slot >= 0
        cs = plsc.cumsum(mask.astype(jnp.int32))
        pos = ptr + cs - 1
        row = lax.shift_right_logical(pos, 7)
        col = lax.bitwise_and(pos, 127)
        plsc.store_scatter(slots2, [row, col], slot, mask=mask)
        plsc.store_scatter(srcs2, [row, col], svec, mask=mask)
        return ptr + jnp.take(cs, lane15)
    with jax.named_scope("filter"):
        kvec = lax.fori_loop(0, E_TILE // 16, filt, jnp.zeros((16,), jnp.int32))
    k = jnp.max(kvec)

    for m in range(CHUNK // 16):
        pos = k + m * 16 + lax.iota(jnp.int32, 16)
        row = lax.shift_right_logical(pos, 7)
        col = lax.bitwise_and(pos, 127)
        plsc.store_scatter(slots2, [row, col],
                           jnp.full((16,), DUMP_SLOT, jnp.int32))
        plsc.store_scatter(srcs2, [row, col], jnp.zeros((16,), jnp.int32))

    plsc.subcore_barrier()

    n_chunks = jnp.maximum(lax.shift_right_logical(k + CHUNK - 1, 7), 1)
    pltpu.async_copy(nf_hbm.at[srcs2.at[0]], rows_v0, gsem0)

    def step(p, carry):
        j = p * 2
        pltpu.make_async_copy(nf_hbm.at[srcs2.at[j]], rows_v0, gsem0).wait()

        @pl.when(j + 1 < n_chunks)
        def _():
            pltpu.async_copy(nf_hbm.at[srcs2.at[j + 1]], rows_v1, gsem1)
        pltpu.sync_copy(rows_v0, agg_sh.at[slots2.at[j]], add=True)

        @pl.when(j + 1 < n_chunks)
        def _():
            pltpu.make_async_copy(
                nf_hbm.at[srcs2.at[j + 1]], rows_v1, gsem1).wait()

            @pl.when(j + 2 < n_chunks)
            def _():
                pltpu.async_copy(nf_hbm.at[srcs2.at[j + 2]], rows_v0, gsem0)
            pltpu.sync_copy(rows_v1, agg_sh.at[slots2.at[j + 1]], add=True)
        return carry
    with jax.named_scope("mainloop"):
        lax.fori_loop(0, lax.shift_right_logical(n_chunks + 1, 1), step, 0)

    plsc.subcore_barrier()

    with jax.named_scope("readout"):
        pltpu.sync_copy(tgt_hbm.at[pl.ds(s * 64, 64)], tgt_v.at[pl.ds(0, 64)])
    for m in range(4):
        tv = tgt_v[pl.ds(m * 16, 16)]
        tgt_v[pl.ds(m * 16, 16)] = plsc.load_gather(inv_v, [tv])
    pltpu.sync_copy(agg_sh.at[tgt_v.at[pl.ds(0, 64)]], ob_v)
    pltpu.sync_copy(ob_v, o_hbm.at[c, pl.ds(s * 64, 64)])


def _combine_body(a_ref, o_ref):
    o_ref[...] = a_ref[0] + a_ref[1]


@jax.jit
def _sc_agg(nf, src, dst, tgt):
    mesh = plsc.VectorSubcoreMesh(core_axis_name="c", subcore_axis_name="s")
    f = pl.kernel(
        _sc_body,
        out_type=jax.ShapeDtypeStruct((NC, N_TARGETS, D_FEAT), jnp.float32),
        mesh=mesh,
        compiler_params=pltpu.CompilerParams(needs_layout_passes=False),
        scratch_types=[
            pltpu.VMEM((INV_ROWS,), jnp.int32),
            pltpu.VMEM((E_TILE,), jnp.int32),
            pltpu.VMEM((E_TILE,), jnp.int32),
            pltpu.VMEM((SL_ROWS, CHUNK), jnp.int32),
            pltpu.VMEM((SL_ROWS, CHUNK), jnp.int32),
            pltpu.VMEM((CHUNK, D_FEAT), jnp.float32),
            pltpu.VMEM((CHUNK, D_FEAT), jnp.float32),
            pltpu.VMEM((16, D_FEAT), jnp.float32),
            pltpu.VMEM((N_TARGETS,), jnp.int32),
            pltpu.VMEM((64, D_FEAT), jnp.float32),
            pltpu.VMEM_SHARED((AGG_ROWS, D_FEAT), jnp.float32),
            pltpu.VMEM_SHARED((INV_ROWS,), jnp.int32),
            pltpu.SemaphoreType.DMA,
            pltpu.SemaphoreType.DMA,
        ],
    )
    partial = f(nf, src, dst, tgt)
    return pl.pallas_call(
        _combine_body,
        out_shape=jax.ShapeDtypeStruct((N_TARGETS, D_FEAT), jnp.float32),
    )(partial)


def kernel(nf, edge_index, target_node_idx):
    return _sc_agg(nf, edge_index[0], edge_index[1], target_node_idx)


# speedup vs baseline: 1.0244x; 1.0244x over previous
import jax
import jax.numpy as jnp
from jax import lax
from jax.experimental import pallas as pl
from jax.experimental.pallas import tpu as pltpu
from jax.experimental.pallas import tpu_sc as plsc

N_NODES = 10000
N_EDGES = 320000
D_FEAT = 128
N_TARGETS = 1024

NC = 2
NS = 16
CHUNK = 128
E_TILE = N_EDGES // (NC * NS)
DUMP_SLOT = N_TARGETS
AGG_ROWS = 1280
SL_ROWS = 80
INV_ROWS = N_NODES + 16


def _sc_body(nf_hbm, src_hbm, dst_hbm, tgt_hbm, o_hbm,
             inv_v, src_v, dst_v, slots2, srcs2, rows_v0, rows_v1,
             rows_v2, zb_v, tgt_v, ob_v, agg_sh, inv_sh,
             gsem0, gsem1, gsem2):
    c = lax.axis_index("c")
    s = lax.axis_index("s")

    base = (c * NS + s) * E_TILE
    with jax.named_scope("stage_idx"):
        pltpu.sync_copy(src_hbm.at[pl.ds(base, E_TILE)], src_v)
        pltpu.sync_copy(dst_hbm.at[pl.ds(base, E_TILE)], dst_v)

    for i in range(16):
        for k in range(D_FEAT // 16):
            zb_v[i, pl.ds(k * 16, 16)] = jnp.zeros((16,), jnp.float32)
    for k in range(5):
        pltpu.sync_copy(zb_v, agg_sh.at[pl.ds(s * 80 + k * 16, 16)])

    @pl.when(s == 0)
    def _():
        def clr(i, carry):
            inv_v[pl.ds(i * 16, 16)] = jnp.full((16,), -1, jnp.int32)
            return carry
        lax.fori_loop(0, INV_ROWS // 16, clr, 0)
        pltpu.sync_copy(tgt_hbm, tgt_v)
        def bld(j, carry):
            tv = tgt_v[pl.ds(j * 16, 16)]
            vals = lax.iota(jnp.int32, 16) + j * 16
            plsc.store_scatter(inv_v, [tv], vals)
            return carry
        lax.fori_loop(0, N_TARGETS // 16, bld, 0)
        pltpu.sync_copy(inv_v, inv_sh)

    plsc.subcore_barrier()
    with jax.named_scope("inv_fetch"):
        pltpu.sync_copy(inv_sh, inv_v)

    lane15 = jnp.full((16,), 15, jnp.int32)

    def filt(i, ptr):
        dvec = dst_v[pl.ds(i * 16, 16)]
        svec = src_v[pl.ds(i * 16, 16)]
        slot = plsc.load_gather(inv_v, [dvec])
        mask = slot >= 0
        cs = plsc.cumsum(mask.astype(jnp.int32))
        pos = ptr + cs - 1
        row = lax.shift_right_logical(pos, 7)
        col = lax.bitwise_and(pos, 127)
        plsc.store_scatter(slots2, [row, col], slot, mask=mask)
        plsc.store_scatter(srcs2, [row, col], svec, mask=mask)
        return ptr + jnp.take(cs, lane15)
    with jax.named_scope("filter"):
        kvec = lax.fori_loop(0, E_TILE // 16, filt, jnp.zeros((16,), jnp.int32))
    k = jnp.max(kvec)

    for m in range(CHUNK // 16):
        pos = k + m * 16 + lax.iota(jnp.int32, 16)
        row = lax.shift_right_logical(pos, 7)
        col = lax.bitwise_and(pos, 127)
        plsc.store_scatter(slots2, [row, col],
                           jnp.full((16,), DUMP_SLOT, jnp.int32))
        plsc.store_scatter(srcs2, [row, col], jnp.zeros((16,), jnp.int32))

    plsc.subcore_barrier()

    n_chunks = jnp.maximum(lax.shift_right_logical(k + CHUNK - 1, 7), 1)
    bufs = [rows_v0, rows_v1, rows_v2]
    sems = [gsem0, gsem1, gsem2]
    for b in range(2):
        @pl.when(b < n_chunks)
        def _(b=b):
            pltpu.async_copy(nf_hbm.at[srcs2.at[b]], bufs[b], sems[b])

    def step(p, carry):
        j0 = p * 3
        for b in range(3):
            j = j0 + b

            @pl.when(j < n_chunks)
            def _(j=j, b=b):
                pltpu.make_async_copy(
                    nf_hbm.at[srcs2.at[j]], bufs[b], sems[b]).wait()

                @pl.when(j + 2 < n_chunks)
                def _(j=j, b=b):
                    pltpu.async_copy(nf_hbm.at[srcs2.at[j + 2]],
                                     bufs[(b + 2) % 3], sems[(b + 2) % 3])
                pltpu.sync_copy(bufs[b], agg_sh.at[slots2.at[j]], add=True)
        return carry
    with jax.named_scope("mainloop"):
        lax.fori_loop(0, (n_chunks + 2) // 3, step, 0)

    plsc.subcore_barrier()

    with jax.named_scope("readout"):
        pltpu.sync_copy(tgt_hbm.at[pl.ds(s * 64, 64)], tgt_v.at[pl.ds(0, 64)])
    for m in range(4):
        tv = tgt_v[pl.ds(m * 16, 16)]
        tgt_v[pl.ds(m * 16, 16)] = plsc.load_gather(inv_v, [tv])
    pltpu.sync_copy(agg_sh.at[tgt_v.at[pl.ds(0, 64)]], ob_v)
    pltpu.sync_copy(ob_v, o_hbm.at[c, pl.ds(s * 64, 64)])


def _combine_body(a_ref, o_ref):
    o_ref[...] = a_ref[0] + a_ref[1]


@jax.jit
def _sc_agg(nf, src, dst, tgt):
    mesh = plsc.VectorSubcoreMesh(core_axis_name="c", subcore_axis_name="s")
    f = pl.kernel(
        _sc_body,
        out_type=jax.ShapeDtypeStruct((NC, N_TARGETS, D_FEAT), jnp.float32),
        mesh=mesh,
        compiler_params=pltpu.CompilerParams(needs_layout_passes=False),
        scratch_types=[
            pltpu.VMEM((INV_ROWS,), jnp.int32),
            pltpu.VMEM((E_TILE,), jnp.int32),
            pltpu.VMEM((E_TILE,), jnp.int32),
            pltpu.VMEM((SL_ROWS, CHUNK), jnp.int32),
            pltpu.VMEM((SL_ROWS, CHUNK), jnp.int32),
            pltpu.VMEM((CHUNK, D_FEAT), jnp.float32),
            pltpu.VMEM((CHUNK, D_FEAT), jnp.float32),
            pltpu.VMEM((CHUNK, D_FEAT), jnp.float32),
            pltpu.VMEM((16, D_FEAT), jnp.float32),
            pltpu.VMEM((N_TARGETS,), jnp.int32),
            pltpu.VMEM((64, D_FEAT), jnp.float32),
            pltpu.VMEM_SHARED((AGG_ROWS, D_FEAT), jnp.float32),
            pltpu.VMEM_SHARED((INV_ROWS,), jnp.int32),
            pltpu.SemaphoreType.DMA,
            pltpu.SemaphoreType.DMA,
            pltpu.SemaphoreType.DMA,
        ],
    )
    partial = f(nf, src, dst, tgt)
    return pl.pallas_call(
        _combine_body,
        out_shape=jax.ShapeDtypeStruct((N_TARGETS, D_FEAT), jnp.float32),
    )(partial)


def kernel(nf, edge_index, target_node_idx):
    return _sc_agg(nf, edge_index[0], edge_index[1], target_node_idx)


# speedup vs baseline: 1.0800x; 1.0543x over previous
import jax
import jax.numpy as jnp
from jax import lax
from jax.experimental import pallas as pl
from jax.experimental.pallas import tpu as pltpu
from jax.experimental.pallas import tpu_sc as plsc

N_NODES = 10000
N_EDGES = 320000
D_FEAT = 128
N_TARGETS = 1024

NC = 2
NS = 16
CHUNK = 128
E_TILE = N_EDGES // (NC * NS)
DUMP_SLOT = N_TARGETS
AGG_ROWS = 1280
SL_ROWS = 80
NF_SH_ROWS = 5120
INV_ROWS = N_NODES + 16


def _sc_body(nf_hbm, src_hbm, dst_hbm, tgt_hbm, o_hbm,
             inv_v, src_v, dst_v, slots2, srcs2, rows_v0,
             zb_v, tgt_v, ob_v, agg_sh, inv_sh, nf_sh):
    c = lax.axis_index("c")
    s = lax.axis_index("s")

    base = (c * NS + s) * E_TILE
    with jax.named_scope("stage_idx"):
        pltpu.sync_copy(src_hbm.at[pl.ds(base, E_TILE)], src_v)
        pltpu.sync_copy(dst_hbm.at[pl.ds(base, E_TILE)], dst_v)

    with jax.named_scope("stage_nf"):
        pltpu.sync_copy(nf_hbm.at[pl.ds(s * 320, 320)],
                        nf_sh.at[pl.ds(s * 320, 320)])

    for i in range(16):
        for k in range(D_FEAT // 16):
            zb_v[i, pl.ds(k * 16, 16)] = jnp.zeros((16,), jnp.float32)
    for k in range(5):
        pltpu.sync_copy(zb_v, agg_sh.at[pl.ds(s * 80 + k * 16, 16)])

    @pl.when(s == 0)
    def _():
        def clr(i, carry):
            inv_v[pl.ds(i * 16, 16)] = jnp.full((16,), -1, jnp.int32)
            return carry
        lax.fori_loop(0, INV_ROWS // 16, clr, 0)
        pltpu.sync_copy(tgt_hbm, tgt_v)
        def bld(j, carry):
            tv = tgt_v[pl.ds(j * 16, 16)]
            vals = lax.iota(jnp.int32, 16) + j * 16
            plsc.store_scatter(inv_v, [tv], vals)
            return carry
        lax.fori_loop(0, N_TARGETS // 16, bld, 0)
        pltpu.sync_copy(inv_v, inv_sh)

    plsc.subcore_barrier()
    with jax.named_scope("inv_fetch"):
        pltpu.sync_copy(inv_sh, inv_v)

    lane15 = jnp.full((16,), 15, jnp.int32)

    NSL = SL_ROWS * CHUNK

    def filt(i, ptrs):
        ptrl, ptrh = ptrs
        dvec = dst_v[pl.ds(i * 16, 16)]
        svec = src_v[pl.ds(i * 16, 16)]
        slot = plsc.load_gather(inv_v, [dvec])
        mask = slot >= 0
        is_lo = svec < NF_SH_ROWS
        lowm = jnp.logical_and(mask, is_lo)
        highm = jnp.logical_and(mask, jnp.logical_not(is_lo))
        csl = plsc.cumsum(lowm.astype(jnp.int32))
        csh = plsc.cumsum(highm.astype(jnp.int32))
        posl = ptrl + csl - 1
        posh = NSL - ptrh - csh
        rowl = lax.shift_right_logical(posl, 7)
        coll = lax.bitwise_and(posl, 127)
        rowh = lax.shift_right_logical(posh, 7)
        colh = lax.bitwise_and(posh, 127)
        plsc.store_scatter(slots2, [rowl, coll], slot, mask=lowm)
        plsc.store_scatter(srcs2, [rowl, coll], svec, mask=lowm)
        plsc.store_scatter(slots2, [rowh, colh], slot, mask=highm)
        plsc.store_scatter(srcs2, [rowh, colh], svec, mask=highm)
        return (ptrl + jnp.take(csl, lane15), ptrh + jnp.take(csh, lane15))
    with jax.named_scope("filter"):
        klvec, khvec = lax.fori_loop(
            0, E_TILE // 16, filt,
            (jnp.zeros((16,), jnp.int32), jnp.zeros((16,), jnp.int32)))
    kl = jnp.max(klvec)
    kh = jnp.max(khvec)

    for m in range(CHUNK // 16):
        pos = kl + m * 16 + lax.iota(jnp.int32, 16)
        row = lax.shift_right_logical(pos, 7)
        col = lax.bitwise_and(pos, 127)
        plsc.store_scatter(slots2, [row, col],
                           jnp.full((16,), DUMP_SLOT, jnp.int32))
        plsc.store_scatter(srcs2, [row, col], jnp.zeros((16,), jnp.int32))
    for m in range(CHUNK // 16):
        pos = NSL - kh - 128 + m * 16 + lax.iota(jnp.int32, 16)
        row = lax.shift_right_logical(pos, 7)
        col = lax.bitwise_and(pos, 127)
        plsc.store_scatter(slots2, [row, col],
                           jnp.full((16,), DUMP_SLOT, jnp.int32))
        plsc.store_scatter(srcs2, [row, col], jnp.zeros((16,), jnp.int32))

    plsc.subcore_barrier()

    n_lo = lax.shift_right_logical(kl + CHUNK - 1, 7)
    n_hi = lax.shift_right_logical(kh + CHUNK - 1, 7)

    def step_lo(j, carry):
        pltpu.sync_copy(nf_sh.at[srcs2.at[j]], rows_v0)
        pltpu.sync_copy(rows_v0, agg_sh.at[slots2.at[j]], add=True)
        return carry

    def step_hi(j, carry):
        pltpu.sync_copy(nf_hbm.at[srcs2.at[j]], rows_v0)
        pltpu.sync_copy(rows_v0, agg_sh.at[slots2.at[j]], add=True)
        return carry
    with jax.named_scope("mainloop"):
        lax.fori_loop(0, n_lo, step_lo, 0)
        lax.fori_loop(SL_ROWS - n_hi, SL_ROWS, step_hi, 0)

    plsc.subcore_barrier()

    with jax.named_scope("readout"):
        pltpu.sync_copy(tgt_hbm.at[pl.ds(s * 64, 64)], tgt_v.at[pl.ds(0, 64)])
    for m in range(4):
        tv = tgt_v[pl.ds(m * 16, 16)]
        tgt_v[pl.ds(m * 16, 16)] = plsc.load_gather(inv_v, [tv])
    pltpu.sync_copy(agg_sh.at[tgt_v.at[pl.ds(0, 64)]], ob_v)
    pltpu.sync_copy(ob_v, o_hbm.at[c, pl.ds(s * 64, 64)])


def _combine_body(a_ref, o_ref):
    o_ref[...] = a_ref[0] + a_ref[1]


@jax.jit
def _sc_agg(nf, src, dst, tgt):
    mesh = plsc.VectorSubcoreMesh(core_axis_name="c", subcore_axis_name="s")
    f = pl.kernel(
        _sc_body,
        out_type=jax.ShapeDtypeStruct((NC, N_TARGETS, D_FEAT), jnp.float32),
        mesh=mesh,
        compiler_params=pltpu.CompilerParams(needs_layout_passes=False, use_tc_tiling_on_sc=False),
        scratch_types=[
            pltpu.VMEM((INV_ROWS,), jnp.int32),
            pltpu.VMEM((E_TILE,), jnp.int32),
            pltpu.VMEM((E_TILE,), jnp.int32),
            pltpu.VMEM((SL_ROWS, CHUNK), jnp.int32),
            pltpu.VMEM((SL_ROWS, CHUNK), jnp.int32),
            pltpu.VMEM((CHUNK, D_FEAT), jnp.float32),
            pltpu.VMEM((16, D_FEAT), jnp.float32),
            pltpu.VMEM((N_TARGETS,), jnp.int32),
            pltpu.VMEM((64, D_FEAT), jnp.float32),
            pltpu.VMEM_SHARED((AGG_ROWS, D_FEAT), jnp.float32),
            pltpu.VMEM_SHARED((INV_ROWS,), jnp.int32),
            pltpu.VMEM_SHARED((NF_SH_ROWS, D_FEAT), jnp.float32),
        ],
    )
    partial = f(nf, src, dst, tgt)
    return pl.pallas_call(
        _combine_body,
        out_shape=jax.ShapeDtypeStruct((N_TARGETS, D_FEAT), jnp.float32),
    )(partial)


def kernel(nf, edge_index, target_node_idx):
    return _sc_agg(nf, edge_index[0], edge_index[1], target_node_idx)


# speedup vs baseline: 1.0809x; 1.0009x over previous
import jax
import jax.numpy as jnp
from jax import lax
from jax.experimental import pallas as pl
from jax.experimental.pallas import tpu as pltpu
from jax.experimental.pallas import tpu_sc as plsc

N_NODES = 10000
N_EDGES = 320000
D_FEAT = 128
N_TARGETS = 1024

NC = 2
NS = 16
CHUNK = 128
E_TILE = N_EDGES // (NC * NS)
DUMP_SLOT = N_TARGETS
AGG_ROWS = 1280
SL_ROWS = 80
NF_SH_ROWS = 5120
INV_ROWS = N_NODES + 16


def _sc_body(nf_hbm, src_hbm, dst_hbm, tgt_hbm, o_hbm,
             inv_v, src_v, dst_v, slots2, srcs2, rows_v0,
             zb_v, tgt_v, ob_v, agg_sh, inv_sh, nf_sh):
    c = lax.axis_index("c")
    s = lax.axis_index("s")

    base = (c * NS + s) * E_TILE
    with jax.named_scope("stage_idx"):
        pltpu.sync_copy(src_hbm.at[pl.ds(base, E_TILE)], src_v)
        pltpu.sync_copy(dst_hbm.at[pl.ds(base, E_TILE)], dst_v)

    with jax.named_scope("stage_nf"):
        pltpu.sync_copy(nf_hbm.at[pl.ds(s * 320, 320)],
                        nf_sh.at[pl.ds(s * 320, 320)])

    for i in range(16):
        for k in range(D_FEAT // 16):
            zb_v[i, pl.ds(k * 16, 16)] = jnp.zeros((16,), jnp.float32)
    for k in range(5):
        pltpu.sync_copy(zb_v, agg_sh.at[pl.ds(s * 80 + k * 16, 16)])

    @pl.when(s == 0)
    def _():
        def clr(i, carry):
            inv_v[pl.ds(i * 16, 16)] = jnp.full((16,), -1, jnp.int32)
            return carry
        lax.fori_loop(0, INV_ROWS // 16, clr, 0)
        pltpu.sync_copy(tgt_hbm, tgt_v)
        def bld(j, carry):
            tv = tgt_v[pl.ds(j * 16, 16)]
            vals = lax.iota(jnp.int32, 16) + j * 16
            plsc.store_scatter(inv_v, [tv], vals)
            return carry
        lax.fori_loop(0, N_TARGETS // 16, bld, 0)
        pltpu.sync_copy(inv_v, inv_sh)

    plsc.subcore_barrier()
    with jax.named_scope("inv_fetch"):
        pltpu.sync_copy(inv_sh, inv_v)

    lane15 = jnp.full((16,), 15, jnp.int32)

    NSL = SL_ROWS * CHUNK

    def filt1(i, ptrl, ptrh):
        dvec = dst_v[pl.ds(i * 16, 16)]
        svec = src_v[pl.ds(i * 16, 16)]
        slot = plsc.load_gather(inv_v, [dvec])
        mask = slot >= 0
        is_lo = svec < NF_SH_ROWS
        lowm = jnp.logical_and(mask, is_lo)
        highm = jnp.logical_and(mask, jnp.logical_not(is_lo))
        csl = plsc.cumsum(lowm.astype(jnp.int32))
        csh = plsc.cumsum(highm.astype(jnp.int32))
        posl = ptrl + csl - 1
        posh = NSL - ptrh - csh
        rowl = lax.shift_right_logical(posl, 7)
        coll = lax.bitwise_and(posl, 127)
        rowh = lax.shift_right_logical(posh, 7)
        colh = lax.bitwise_and(posh, 127)
        plsc.store_scatter(slots2, [rowl, coll], slot, mask=lowm)
        plsc.store_scatter(srcs2, [rowl, coll], svec, mask=lowm)
        plsc.store_scatter(slots2, [rowh, colh], slot, mask=highm)
        plsc.store_scatter(srcs2, [rowh, colh], svec, mask=highm)
        return (ptrl + jnp.take(csl, lane15), ptrh + jnp.take(csh, lane15))

    def filt(p, ptrs):
        ptrl, ptrh = ptrs
        ptrl, ptrh = filt1(p * 2, ptrl, ptrh)
        return filt1(p * 2 + 1, ptrl, ptrh)
    with jax.named_scope("filter"):
        klvec, khvec = lax.fori_loop(
            0, E_TILE // 32, filt,
            (jnp.zeros((16,), jnp.int32), jnp.zeros((16,), jnp.int32)))
        klvec, khvec = filt1(E_TILE // 16 - 1, klvec, khvec)
    kl = jnp.max(klvec)
    kh = jnp.max(khvec)

    for m in range(CHUNK // 16):
        pos = kl + m * 16 + lax.iota(jnp.int32, 16)
        row = lax.shift_right_logical(pos, 7)
        col = lax.bitwise_and(pos, 127)
        plsc.store_scatter(slots2, [row, col],
                           jnp.full((16,), DUMP_SLOT, jnp.int32))
        plsc.store_scatter(srcs2, [row, col], jnp.zeros((16,), jnp.int32))
    for m in range(CHUNK // 16):
        pos = NSL - kh - 128 + m * 16 + lax.iota(jnp.int32, 16)
        row = lax.shift_right_logical(pos, 7)
        col = lax.bitwise_and(pos, 127)
        plsc.store_scatter(slots2, [row, col],
                           jnp.full((16,), DUMP_SLOT, jnp.int32))
        plsc.store_scatter(srcs2, [row, col], jnp.zeros((16,), jnp.int32))

    plsc.subcore_barrier()

    n_lo = lax.shift_right_logical(kl + CHUNK - 1, 7)
    n_hi = lax.shift_right_logical(kh + CHUNK - 1, 7)

    def step_lo(j, carry):
        pltpu.sync_copy(nf_sh.at[srcs2.at[j]], rows_v0)
        pltpu.sync_copy(rows_v0, agg_sh.at[slots2.at[j]], add=True)
        return carry

    def step_hi(j, carry):
        pltpu.sync_copy(nf_hbm.at[srcs2.at[j]], rows_v0)
        pltpu.sync_copy(rows_v0, agg_sh.at[slots2.at[j]], add=True)
        return carry
    with jax.named_scope("mainloop"):
        lax.fori_loop(0, n_lo, step_lo, 0)
        lax.fori_loop(SL_ROWS - n_hi, SL_ROWS, step_hi, 0)

    plsc.subcore_barrier()

    with jax.named_scope("readout"):
        pltpu.sync_copy(tgt_hbm.at[pl.ds(s * 64, 64)], tgt_v.at[pl.ds(0, 64)])
    for m in range(4):
        tv = tgt_v[pl.ds(m * 16, 16)]
        tgt_v[pl.ds(m * 16, 16)] = plsc.load_gather(inv_v, [tv])
    pltpu.sync_copy(agg_sh.at[tgt_v.at[pl.ds(0, 64)]], ob_v)
    pltpu.sync_copy(ob_v, o_hbm.at[c, pl.ds(s * 64, 64)])


def _combine_body(a_ref, o_ref):
    o_ref[...] = a_ref[0] + a_ref[1]


@jax.jit
def _sc_agg(nf, src, dst, tgt):
    mesh = plsc.VectorSubcoreMesh(core_axis_name="c", subcore_axis_name="s")
    f = pl.kernel(
        _sc_body,
        out_type=jax.ShapeDtypeStruct((NC, N_TARGETS, D_FEAT), jnp.float32),
        mesh=mesh,
        compiler_params=pltpu.CompilerParams(needs_layout_passes=False, use_tc_tiling_on_sc=False),
        scratch_types=[
            pltpu.VMEM((INV_ROWS,), jnp.int32),
            pltpu.VMEM((E_TILE,), jnp.int32),
            pltpu.VMEM((E_TILE,), jnp.int32),
            pltpu.VMEM((SL_ROWS, CHUNK), jnp.int32),
            pltpu.VMEM((SL_ROWS, CHUNK), jnp.int32),
            pltpu.VMEM((CHUNK, D_FEAT), jnp.float32),
            pltpu.VMEM((16, D_FEAT), jnp.float32),
            pltpu.VMEM((N_TARGETS,), jnp.int32),
            pltpu.VMEM((64, D_FEAT), jnp.float32),
            pltpu.VMEM_SHARED((AGG_ROWS, D_FEAT), jnp.float32),
            pltpu.VMEM_SHARED((INV_ROWS,), jnp.int32),
            pltpu.VMEM_SHARED((NF_SH_ROWS, D_FEAT), jnp.float32),
        ],
    )
    partial = f(nf, src, dst, tgt)
    return pl.pallas_call(
        _combine_body,
        out_shape=jax.ShapeDtypeStruct((N_TARGETS, D_FEAT), jnp.float32),
    )(partial)


def kernel(nf, edge_index, target_node_idx):
    return _sc_agg(nf, edge_index[0], edge_index[1], target_node_idx)
